# Initial kernel scaffold; baseline (speedup 1.0000x reference)
import jax
import jax.numpy as jnp
from jax.experimental import pallas as pl


def kernel(x, edge_index, batch, W1a, b1a, W1b, b1b, W2a, b2a, W2b, b2b, Wg, bg, Wo, bo):
    raise NotImplementedError("write your pallas kernel here")


# speedup vs baseline: 7.3346x; 7.3346x over previous
import functools

import jax
import jax.numpy as jnp
from jax import lax
from jax.experimental import pallas as pl
from jax.experimental.pallas import tpu as pltpu
from jax.experimental.pallas import tpu_sc as plsc

N_NODES = 10000
N_PAD = 10240
N_EDGES = 320000
HID = 64
N_GRAPHS = 128

_NC, _NS = 2, 16
_NW = _NC * _NS
_EPT = N_EDGES // _NW
_CH = 128
_NFULL = _EPT // _CH
_TAIL = _EPT - _NFULL * _CH
_RPT = N_PAD // _NS


def _edge_agg(y, src, dst):
    mesh = plsc.VectorSubcoreMesh(core_axis_name="c", subcore_axis_name="s")

    @functools.partial(
        pl.kernel,
        mesh=mesh,
        compiler_params=pltpu.CompilerParams(use_tc_tiling_on_sc=False),
        out_type=jax.ShapeDtypeStruct((_NC, N_PAD, HID), jnp.float32),
        scratch_types=[
            pltpu.VMEM((_CH,), jnp.int32),
            pltpu.VMEM((_CH,), jnp.int32),
            pltpu.VMEM((_TAIL,), jnp.int32),
            pltpu.VMEM((_TAIL,), jnp.int32),
            pltpu.VMEM((_CH, HID), jnp.float32),
            pltpu.VMEM((_TAIL, HID), jnp.float32),
            pltpu.VMEM((_RPT, HID), jnp.float32),
            pltpu.VMEM_SHARED((N_PAD, HID), jnp.float32),
            pltpu.SemaphoreType.DMA,
        ],
    )
    def agg(y_hbm, src_hbm, dst_hbm, out_hbm,
            sidx, didx, sidx_t, didx_t, rows, rows_t, zbuf, acc, sem):
        c = lax.axis_index("c")
        s = lax.axis_index("s")
        tid = c * _NS + s

        zeros16 = jnp.zeros((16,), jnp.float32)

        def zrow(r, _):
            for j in range(HID // 16):
                zbuf[r, pl.ds(j * 16, 16)] = zeros16
            return ()

        lax.fori_loop(0, _RPT, zrow, ())
        pltpu.sync_copy(zbuf, acc.at[pl.ds(s * _RPT, _RPT)])
        plsc.subcore_barrier()

        base = tid * _EPT

        def body(k, _):
            off = base + k * _CH
            pltpu.sync_copy(src_hbm.at[pl.ds(off, _CH)], sidx)
            pltpu.sync_copy(dst_hbm.at[pl.ds(off, _CH)], didx)
            pltpu.async_copy(y_hbm.at[sidx], rows, sem).wait()
            pltpu.sync_copy(rows, acc.at[didx], add=True)
            return ()

        lax.fori_loop(0, _NFULL, body, ())

        off = base + _NFULL * _CH
        pltpu.sync_copy(src_hbm.at[pl.ds(off, _TAIL)], sidx_t)
        pltpu.sync_copy(dst_hbm.at[pl.ds(off, _TAIL)], didx_t)
        pltpu.async_copy(y_hbm.at[sidx_t], rows_t, sem).wait()
        pltpu.sync_copy(rows_t, acc.at[didx_t], add=True)

        plsc.subcore_barrier()
        pltpu.sync_copy(acc.at[pl.ds(s * _RPT, _RPT)],
                        out_hbm.at[c].at[pl.ds(s * _RPT, _RPT)])

    return agg(y, src, dst)


_BM = 1000


def _proj_body(x_ref, w_ref, o_ref):
    o_ref[...] = jnp.dot(x_ref[...], w_ref[...],
                         preferred_element_type=jnp.float32)


def _proj(x, w):
    m, k = x.shape
    n = w.shape[1]
    return pl.pallas_call(
        _proj_body,
        grid=(N_NODES // _BM,),
        in_specs=[
            pl.BlockSpec((_BM, k), lambda i: (i, 0)),
            pl.BlockSpec((k, n), lambda i: (0, 0)),
        ],
        out_specs=pl.BlockSpec((_BM, n), lambda i: (i, 0)),
        out_shape=jax.ShapeDtypeStruct((N_PAD, n), jnp.float32),
    )(x, w)


def _mid_body(y1_ref, p_ref, b1a_ref, w1b_ref, b1b_ref, w2a_ref, o_ref):
    u = jnp.maximum(y1_ref[...] + p_ref[0] + p_ref[1] + b1a_ref[...], 0.0)
    h = jnp.maximum(
        jnp.dot(u, w1b_ref[...], preferred_element_type=jnp.float32)
        + b1b_ref[...], 0.0)
    o_ref[...] = jnp.dot(h, w2a_ref[...], preferred_element_type=jnp.float32)


def _mid(y1, p, b1a, w1b, b1b, w2a):
    return pl.pallas_call(
        _mid_body,
        grid=(N_NODES // _BM,),
        in_specs=[
            pl.BlockSpec((_BM, HID), lambda i: (i, 0)),
            pl.BlockSpec((_NC, _BM, HID), lambda i: (0, i, 0)),
            pl.BlockSpec((1, HID), lambda i: (0, 0)),
            pl.BlockSpec((HID, HID), lambda i: (0, 0)),
            pl.BlockSpec((1, HID), lambda i: (0, 0)),
            pl.BlockSpec((HID, HID), lambda i: (0, 0)),
        ],
        out_specs=pl.BlockSpec((_BM, HID), lambda i: (i, 0)),
        out_shape=jax.ShapeDtypeStruct((N_PAD, HID), jnp.float32),
    )(y1, p, b1a, w1b, b1b, w2a)


def _tail_body(y2_ref, p_ref, b2a_ref, w2b_ref, b2b_ref, batch_ref,
               wg_ref, bg_ref, wo_ref, bo_ref, g_ref, o_ref):
    i = pl.program_id(0)
    ng = pl.num_programs(0)
    v = jnp.maximum(y2_ref[...] + p_ref[0] + p_ref[1] + b2a_ref[...], 0.0)
    h2 = jnp.maximum(
        jnp.dot(v, w2b_ref[...], preferred_element_type=jnp.float32)
        + b2b_ref[...], 0.0)
    ids = batch_ref[pl.ds(i, 1), :]
    onehot_t = (jnp.broadcast_to(ids, (N_GRAPHS, _BM))
                == lax.broadcasted_iota(jnp.int32, (N_GRAPHS, _BM), 0)
                ).astype(jnp.float32)
    gpart = lax.dot_general(onehot_t, h2, (((1,), (0,)), ((), ())),
                            preferred_element_type=jnp.float32)

    @pl.when(i == 0)
    def _init():
        g_ref[...] = gpart

    @pl.when(i > 0)
    def _accum():
        g_ref[...] += gpart

    @pl.when(i == ng - 1)
    def _head():
        g = g_ref[...]
        t = jnp.maximum(
            jnp.dot(g, wg_ref[...], preferred_element_type=jnp.float32)
            + bg_ref[...], 0.0)
        o_ref[...] = (jnp.dot(t, wo_ref[...],
                              preferred_element_type=jnp.float32)
                      + bo_ref[...])


def _tail_stage(y2, p, b2a, w2b, b2b, batch2d, wg, bg, wo, bo):
    out_dim = wo.shape[1]
    nb = N_NODES // _BM
    _, out = pl.pallas_call(
        _tail_body,
        grid=(nb,),
        in_specs=[
            pl.BlockSpec((_BM, HID), lambda i: (i, 0)),
            pl.BlockSpec((_NC, _BM, HID), lambda i: (0, i, 0)),
            pl.BlockSpec((1, HID), lambda i: (0, 0)),
            pl.BlockSpec((HID, HID), lambda i: (0, 0)),
            pl.BlockSpec((1, HID), lambda i: (0, 0)),
            pl.BlockSpec((nb, _BM), lambda i: (0, 0)),
            pl.BlockSpec((HID, HID), lambda i: (0, 0)),
            pl.BlockSpec((1, HID), lambda i: (0, 0)),
            pl.BlockSpec((HID, out_dim), lambda i: (0, 0)),
            pl.BlockSpec((1, out_dim), lambda i: (0, 0)),
        ],
        out_specs=[
            pl.BlockSpec((N_GRAPHS, HID), lambda i: (0, 0)),
            pl.BlockSpec((N_GRAPHS, out_dim), lambda i: (0, 0)),
        ],
        out_shape=[
            jax.ShapeDtypeStruct((N_GRAPHS, HID), jnp.float32),
            jax.ShapeDtypeStruct((N_GRAPHS, out_dim), jnp.float32),
        ],
    )(y2, p, b2a, w2b, b2b, batch2d, wg, bg, wo, bo)
    return out


def kernel(x, edge_index, batch, W1a, b1a, W1b, b1b, W2a, b2a, W2b, b2b,
           Wg, bg, Wo, bo):
    src = edge_index[0].astype(jnp.int32)
    dst = edge_index[1].astype(jnp.int32)
    batch2d = batch.astype(jnp.int32).reshape(N_NODES // _BM, _BM)

    y1 = _proj(x, W1a)
    p1 = _edge_agg(y1, src, dst)
    y2 = _mid(y1, p1, b1a.reshape(1, -1), W1b, b1b.reshape(1, -1), W2a)
    p2 = _edge_agg(y2, src, dst)
    return _tail_stage(y2, p2, b2a.reshape(1, -1), W2b, b2b.reshape(1, -1),
                       batch2d, Wg, bg.reshape(1, -1), Wo, bo.reshape(1, -1))
